# Initial kernel scaffold; baseline (speedup 1.0000x reference)
import jax
import jax.numpy as jnp
from jax.experimental import pallas as pl
from jax.experimental.pallas import tpu as pltpu

IN_DIM = 128
OUT_DIM = 256
MEM_LEN = 1024
BATCH = 512
BETA = 200.0
EPS = 1e-8
MC = 8


def _mem_stream_kernel(x_ref, mem_ref, md_ref, w_ref, b_ref, mean_ref, std_ref,
                       scores_ref, newmem_ref, newmd_ref, nmean_ref, nstd_ref,
                       d_scr, enc_scr):
    mean = mean_ref[...]
    std = std_ref[...]
    deg = std < EPS
    safe_std = jnp.where(deg, 1.0, std)
    xn = (x_ref[...] - mean) / safe_std
    xn = jnp.where(deg, 0.0, xn)

    enc = jax.lax.dot_general(xn, w_ref[...], (((1,), (1,)), ((), ())))
    enc = jnp.maximum(enc + b_ref[...], 0.0)
    enc_scr[...] = enc

    def dist_body(i, _):
        mc = mem_ref[pl.ds(i * MC, MC), :]
        diff = jnp.abs(enc_scr[...][None, :, :] - mc[:, None, :])
        d_scr[pl.ds(i * MC, MC), :] = jnp.sum(diff, axis=-1)
        return 0
    jax.lax.fori_loop(0, MEM_LEN // MC, dist_body, 0)

    iota_m = jax.lax.broadcasted_iota(jnp.float32, (MEM_LEN, BATCH), 0)
    big = jnp.float32(MEM_LEN)
    m1 = jnp.min(d_scr[...], axis=0, keepdims=True)
    j1 = jnp.min(jnp.where(d_scr[...] == m1, iota_m, big), axis=0, keepdims=True)
    d_scr[...] = jnp.where(iota_m == j1, jnp.inf, d_scr[...])
    m2 = jnp.min(d_scr[...], axis=0, keepdims=True)
    j2 = jnp.min(jnp.where(d_scr[...] == m2, iota_m, big), axis=0, keepdims=True)
    d_scr[...] = jnp.where(iota_m == j2, jnp.inf, d_scr[...])
    m3 = jnp.min(d_scr[...], axis=0, keepdims=True)
    scores = (m1 + m2 + m3) / 3.0
    scores_ref[...] = scores

    maskf = (scores <= BETA).astype(jnp.float32)
    bi = jax.lax.broadcasted_iota(jnp.float32, (BATCH, BATCH), 0)
    bj = jax.lax.broadcasted_iota(jnp.float32, (BATCH, BATCH), 1)
    tri = (bi <= bj).astype(jnp.float32)
    csum = jax.lax.dot_general(maskf, tri, (((1,), (0,)), ((), ())))
    n_upd = csum[0:1, BATCH - 1:BATCH]
    pos = jnp.where(maskf > 0.0, csum - 1.0, big)

    jrow = jax.lax.broadcasted_iota(jnp.float32, (MEM_LEN, BATCH), 0)
    sel = (pos == jrow).astype(jnp.float32)
    upd_rows = (jax.lax.broadcasted_iota(jnp.float32, (MEM_LEN, 1), 0)
                < n_upd)
    sel_enc = jax.lax.dot_general(sel, enc_scr[...], (((1,), (0,)), ((), ())))
    newmem_ref[...] = jnp.where(upd_rows, sel_enc, mem_ref[...])
    sel_x = jax.lax.dot_general(sel, x_ref[...], (((1,), (0,)), ((), ())))
    newmd_ref[...] = jnp.where(upd_rows, sel_x, md_ref[...])

    md = md_ref[...]
    rowmf = upd_rows.astype(jnp.float32)
    sum_all = jnp.sum(md, axis=0, keepdims=True)
    ssq_all = jnp.sum(md * md, axis=0, keepdims=True)
    sum_repl = jax.lax.dot_general(rowmf, md, (((0,), (0,)), ((), ())))
    ssq_repl = jax.lax.dot_general(rowmf, md * md, (((0,), (0,)), ((), ())))
    xv = x_ref[...]
    xsum = jax.lax.dot_general(maskf, xv, (((1,), (0,)), ((), ())))
    xssq = jax.lax.dot_general(maskf, xv * xv, (((1,), (0,)), ((), ())))
    nsum = sum_all - sum_repl + xsum
    nssq = ssq_all - ssq_repl + xssq
    m = nsum / jnp.float32(MEM_LEN)
    var = (nssq - jnp.float32(MEM_LEN) * m * m) / jnp.float32(MEM_LEN - 1)
    s = jnp.sqrt(jnp.maximum(var, 0.0))
    s = jnp.where(s < EPS, 1.0, s)
    did_upd = n_upd > 0.0
    nmean_ref[...] = jnp.where(did_upd, m, mean)
    nstd_ref[...] = jnp.where(did_upd, s, std)


def kernel(x, memory, mem_data, W_enc, b_enc, mean, std):
    f32 = jnp.float32
    outs = pl.pallas_call(
        _mem_stream_kernel,
        out_shape=[
            jax.ShapeDtypeStruct((1, BATCH), f32),
            jax.ShapeDtypeStruct((MEM_LEN, OUT_DIM), f32),
            jax.ShapeDtypeStruct((MEM_LEN, IN_DIM), f32),
            jax.ShapeDtypeStruct((1, IN_DIM), f32),
            jax.ShapeDtypeStruct((1, IN_DIM), f32),
        ],
        scratch_shapes=[
            pltpu.VMEM((MEM_LEN, BATCH), f32),
            pltpu.VMEM((BATCH, OUT_DIM), f32),
        ],
    )(x, memory, mem_data, W_enc, b_enc.reshape(1, OUT_DIM),
      mean.reshape(1, IN_DIM), std.reshape(1, IN_DIM))
    scores, new_memory, new_mem_data, new_mean, new_std = outs
    return (scores.reshape(BATCH), new_memory, new_mem_data,
            new_mean.reshape(IN_DIM), new_std.reshape(IN_DIM))


# speedup vs baseline: 2.5840x; 2.5840x over previous
import jax
import jax.numpy as jnp
from jax.experimental import pallas as pl
from jax.experimental.pallas import tpu as pltpu

IN_DIM = 128
OUT_DIM = 256
MEM_LEN = 1024
BATCH = 512
BETA = 200.0
EPS = 1e-8
MC = 8


def _mem_stream_kernel(x_ref, mem_ref, md_ref, w_ref, b_ref, mean_ref, std_ref,
                       scores_ref, newmem_ref, newmd_ref, nmean_ref, nstd_ref,
                       d_scr, enc_scr):
    mean = mean_ref[...]
    std = std_ref[...]
    deg = std < EPS
    safe_std = jnp.where(deg, 1.0, std)
    xn = (x_ref[...] - mean) / safe_std
    xn = jnp.where(deg, 0.0, xn)

    enc = jax.lax.dot_general(xn, w_ref[...], (((1,), (1,)), ((), ())))
    enc = jnp.maximum(enc + b_ref[...], 0.0)
    enc_scr[...] = enc

    def dist_body(i, _):
        mc = mem_ref[pl.ds(i * MC, MC), :]
        diff = jnp.abs(enc_scr[...][None, :, :] - mc[:, None, :])
        d_scr[pl.ds(i * MC, MC), :] = jnp.sum(diff, axis=-1)
        return 0
    jax.lax.fori_loop(0, MEM_LEN // MC, dist_body, 0)

    iota_m = jax.lax.broadcasted_iota(jnp.int32, (MEM_LEN, BATCH), 0)
    big = jnp.int32(MEM_LEN)
    m1 = jnp.min(d_scr[...], axis=0, keepdims=True)
    j1 = jnp.min(jnp.where(d_scr[...] == m1, iota_m, big), axis=0, keepdims=True)
    d_scr[...] = jnp.where(iota_m == j1, jnp.inf, d_scr[...])
    m2 = jnp.min(d_scr[...], axis=0, keepdims=True)
    j2 = jnp.min(jnp.where(d_scr[...] == m2, iota_m, big), axis=0, keepdims=True)
    d_scr[...] = jnp.where(iota_m == j2, jnp.inf, d_scr[...])
    m3 = jnp.min(d_scr[...], axis=0, keepdims=True)
    scores = (m1 + m2 + m3) / 3.0
    scores_ref[...] = scores

    maskf = (scores <= BETA).astype(jnp.float32)
    bi = jax.lax.broadcasted_iota(jnp.int32, (BATCH, BATCH), 0)
    bj = jax.lax.broadcasted_iota(jnp.int32, (BATCH, BATCH), 1)
    tri = (bi <= bj).astype(jnp.float32)
    csum = jax.lax.dot_general(maskf, tri, (((1,), (0,)), ((), ())))
    n_upd = csum[0:1, BATCH - 1:BATCH].astype(jnp.int32)
    posi = jnp.where(maskf > 0.0, csum - 1.0, jnp.float32(MEM_LEN)
                     ).astype(jnp.int32)

    sel = (posi == iota_m).astype(jnp.float32)
    upd_rows = (jax.lax.broadcasted_iota(jnp.int32, (MEM_LEN, 1), 0)
                < n_upd)
    sel_enc = jax.lax.dot_general(sel, enc_scr[...], (((1,), (0,)), ((), ())))
    newmem_ref[...] = jnp.where(upd_rows, sel_enc, mem_ref[...])
    sel_x = jax.lax.dot_general(sel, x_ref[...], (((1,), (0,)), ((), ())))
    newmd_ref[...] = jnp.where(upd_rows, sel_x, md_ref[...])

    md = md_ref[...]
    rowmf = upd_rows.astype(jnp.float32)
    sum_all = jnp.sum(md, axis=0, keepdims=True)
    ssq_all = jnp.sum(md * md, axis=0, keepdims=True)
    sum_repl = jax.lax.dot_general(rowmf, md, (((0,), (0,)), ((), ())))
    ssq_repl = jax.lax.dot_general(rowmf, md * md, (((0,), (0,)), ((), ())))
    xv = x_ref[...]
    xsum = jax.lax.dot_general(maskf, xv, (((1,), (0,)), ((), ())))
    xssq = jax.lax.dot_general(maskf, xv * xv, (((1,), (0,)), ((), ())))
    nsum = sum_all - sum_repl + xsum
    nssq = ssq_all - ssq_repl + xssq
    m = nsum / jnp.float32(MEM_LEN)
    var = (nssq - jnp.float32(MEM_LEN) * m * m) / jnp.float32(MEM_LEN - 1)
    s = jnp.sqrt(jnp.maximum(var, 0.0))
    s = jnp.where(s < EPS, 1.0, s)
    did_upd = n_upd > 0
    nmean_ref[...] = jnp.where(did_upd, m, mean)
    nstd_ref[...] = jnp.where(did_upd, s, std)


def kernel(x, memory, mem_data, W_enc, b_enc, mean, std):
    f32 = jnp.float32
    outs = pl.pallas_call(
        _mem_stream_kernel,
        out_shape=[
            jax.ShapeDtypeStruct((1, BATCH), f32),
            jax.ShapeDtypeStruct((MEM_LEN, OUT_DIM), f32),
            jax.ShapeDtypeStruct((MEM_LEN, IN_DIM), f32),
            jax.ShapeDtypeStruct((1, IN_DIM), f32),
            jax.ShapeDtypeStruct((1, IN_DIM), f32),
        ],
        scratch_shapes=[
            pltpu.VMEM((MEM_LEN, BATCH), f32),
            pltpu.VMEM((BATCH, OUT_DIM), f32),
        ],
    )(x, memory, mem_data, W_enc, b_enc.reshape(1, OUT_DIM),
      mean.reshape(1, IN_DIM), std.reshape(1, IN_DIM))
    scores, new_memory, new_mem_data, new_mean, new_std = outs
    return (scores.reshape(BATCH), new_memory, new_mem_data,
            new_mean.reshape(IN_DIM), new_std.reshape(IN_DIM))
